# baseline (speedup 1.0000x reference)
import functools

import jax
import jax.numpy as jnp
from jax import lax
from jax.experimental import pallas as pl
from jax.experimental.pallas import tpu as pltpu


def _sig(x):
    return 1.0 / (1.0 + jnp.exp(-x))


def _gates(g, H):
    i_g = _sig(g[:, 0:H])
    f_g = _sig(g[:, H:2 * H])
    g_g = jnp.tanh(g[:, 2 * H:3 * H])
    o_g = _sig(g[:, 3 * H:4 * H])
    return i_g, f_g, g_g, o_g


def _encoder_kernel(ids_ref, emb_ref, wih0_ref, whh0_ref, b0_ref,
                    w1_ref, b1_ref,
                    out_ref, hn0_ref, cn0_ref, hn1_ref, cn1_ref,
                    xg_ref, lhs1_ref,
                    *, seq_len, batch, hidden, vocab, unroll):
    S, B, H, V = seq_len, batch, hidden, vocab

    emb = emb_ref[...]
    wih0 = wih0_ref[...]
    b0 = b0_ref[...]
    rows = S * B
    chunk = 128 if rows % 128 == 0 else rows
    for mc in range(rows // chunk):
        ids_c = ids_ref[mc * chunk:(mc + 1) * chunk, :]
        iota = lax.broadcasted_iota(jnp.int32, (chunk, V), 1)
        oh = (ids_c == iota).astype(jnp.float32)
        er = jnp.dot(oh, emb, preferred_element_type=jnp.float32)
        xg = jnp.dot(er, wih0,
                     preferred_element_type=jnp.float32) + b0
        xg_ref[mc * chunk:(mc + 1) * chunk, :] = xg

    w1 = w1_ref[...]
    whh0 = whh0_ref[...]
    b1 = b1_ref[...]

    g0 = xg_ref[0:B, :]
    i0, f0, gg0, o0 = _gates(g0, H)
    c0n = i0 * gg0
    h0n = o0 * jnp.tanh(c0n)
    cn0_ref[...] = c0n
    hn0_ref[...] = h0n
    lhs1_ref[...] = jnp.concatenate(
        [h0n, jnp.zeros((B, H), jnp.float32)], axis=1)
    cn1_ref[...] = jnp.zeros((B, H), jnp.float32)

    def substep(i):
        a1 = lhs1_ref[...]
        g1 = jnp.dot(a1, w1, preferred_element_type=jnp.float32) + b1
        g0 = jnp.dot(a1[:, 0:H], whh0,
                     preferred_element_type=jnp.float32) + xg_ref[pl.ds(i * B, B), :]

        i0, f0, gg0, o0 = _gates(g0, H)
        c0n = f0 * cn0_ref[...] + i0 * gg0
        h0n = o0 * jnp.tanh(c0n)
        cn0_ref[...] = c0n
        hn0_ref[...] = h0n

        i1, f1, gg1, o1 = _gates(g1, H)
        c1n = f1 * cn1_ref[...] + i1 * gg1
        h1n = o1 * jnp.tanh(c1n)
        cn1_ref[...] = c1n
        out_ref[pl.ds((i - 1) * B, B), :] = h1n

        lhs1_ref[...] = jnp.concatenate([h0n, h1n], axis=1)

    if unroll >= S - 1:
        for i in range(1, S):
            substep(i)
    else:
        n_main = ((S - 1) // unroll) * unroll
        n_peel = (S - 1) - n_main
        for i in range(1, 1 + n_peel):
            substep(i)

        def body(k, carry):
            base = 1 + n_peel + k * unroll
            for o in range(unroll):
                substep(base + o)
            return carry

        lax.fori_loop(0, n_main // unroll, body, 0)

    a1 = lhs1_ref[...]
    g1 = jnp.dot(a1, w1, preferred_element_type=jnp.float32) + b1
    i1, f1, gg1, o1 = _gates(g1, H)
    c1n = f1 * cn1_ref[...] + i1 * gg1
    h1n = o1 * jnp.tanh(c1n)
    cn1_ref[...] = c1n
    hn1_ref[...] = h1n
    out_ref[pl.ds((S - 1) * B, B), :] = h1n


def kernel(ids, embedding, w_ih_0, w_hh_0, b_ih_0, b_hh_0,
           w_ih_1, w_hh_1, b_ih_1, b_hh_1):
    B, S = ids.shape
    V, E = embedding.shape
    H = w_hh_0.shape[1]

    ids_col = jnp.transpose(ids).reshape(S * B, 1).astype(jnp.int32)
    wih0_t = jnp.transpose(w_ih_0)
    whh0_t = jnp.transpose(w_hh_0)
    w1_t = jnp.concatenate(
        [jnp.transpose(w_ih_1), jnp.transpose(w_hh_1)], axis=0)
    b0 = (b_ih_0 + b_hh_0).reshape(1, 4 * H)
    b1 = (b_ih_1 + b_hh_1).reshape(1, 4 * H)

    out_shapes = (
        jax.ShapeDtypeStruct((S * B, H), jnp.float32),
        jax.ShapeDtypeStruct((B, H), jnp.float32),
        jax.ShapeDtypeStruct((B, H), jnp.float32),
        jax.ShapeDtypeStruct((B, H), jnp.float32),
        jax.ShapeDtypeStruct((B, H), jnp.float32),
    )

    def full(x):
        n = len(x.shape)
        return pl.BlockSpec(x.shape, lambda: (0,) * n)

    inputs = (ids_col, embedding, wih0_t, whh0_t, b0, w1_t, b1)
    out_flat, hn0, cn0, hn1, cn1 = pl.pallas_call(
        functools.partial(_encoder_kernel, seq_len=S, batch=B, hidden=H,
                          vocab=V, unroll=64),
        out_shape=out_shapes,
        in_specs=[full(x) for x in inputs],
        out_specs=tuple(pl.BlockSpec(s.shape, lambda: (0,) * len(s.shape))
                        for s in out_shapes),
        scratch_shapes=[
            pltpu.VMEM((S * B, 4 * H), jnp.float32),
            pltpu.VMEM((B, 2 * H), jnp.float32),
        ],
        compiler_params=pltpu.CompilerParams(
            dimension_semantics=()),
    )(*inputs)

    out = jnp.transpose(out_flat.reshape(S, B, H), (1, 0, 2))
    h_n = jnp.stack([hn0, hn1], axis=0)
    c_n = jnp.stack([cn0, cn1], axis=0)
    return out, (h_n, c_n)


# speedup vs baseline: 1.5247x; 1.5247x over previous
import functools

import jax
import jax.numpy as jnp
from jax import lax
from jax.experimental import pallas as pl
from jax.experimental.pallas import tpu as pltpu


def _sig(x):
    return 1.0 / (1.0 + jnp.exp(-x))


def _gates(g, H):
    i_g = _sig(g[:, 0:H])
    f_g = _sig(g[:, H:2 * H])
    g_g = jnp.tanh(g[:, 2 * H:3 * H])
    o_g = _sig(g[:, 3 * H:4 * H])
    return i_g, f_g, g_g, o_g


def _encoder_kernel(ids_ref, emb_ref, wih0_ref, whh0_ref, bih0_ref, bhh0_ref,
                    wih1_ref, whh1_ref, bih1_ref, bhh1_ref,
                    out_ref, hn_ref, cn_ref,
                    xg_ref, whh0t_ref, w1t_ref,
                    *, seq_len, batch, hidden, vocab):
    S, B, H, V = seq_len, batch, hidden, vocab

    whh0t_ref[...] = jnp.transpose(whh0_ref[...])
    w1t_ref[0:H, :] = jnp.transpose(wih1_ref[...])
    w1t_ref[H:2 * H, :] = jnp.transpose(whh1_ref[...])
    b0 = bih0_ref[...] + bhh0_ref[...]
    b1 = bih1_ref[...] + bhh1_ref[...]

    emb = emb_ref[...]
    wih0t = jnp.transpose(wih0_ref[...])
    rows = S * B
    chunk = 128 if rows % 128 == 0 else rows
    for mc in range(rows // chunk):
        ids_c = ids_ref[mc * chunk:(mc + 1) * chunk, :]
        iota = lax.broadcasted_iota(jnp.int32, (chunk, V), 1)
        oh = (ids_c == iota).astype(jnp.float32)
        er = jnp.dot(oh, emb, preferred_element_type=jnp.float32)
        xg = jnp.dot(er, wih0t,
                     preferred_element_type=jnp.float32) + b0
        xg_ref[mc * chunk:(mc + 1) * chunk, :] = xg

    whh0t = whh0t_ref[...]
    w1t = w1t_ref[...]

    g0 = xg_ref[0:B, :]
    i0, f0, gg0, o0 = _gates(g0, H)
    c0 = i0 * gg0
    h0 = o0 * jnp.tanh(c0)
    h1 = jnp.zeros((B, H), jnp.float32)
    c1 = jnp.zeros((B, H), jnp.float32)

    for i in range(1, S):
        a1 = jnp.concatenate([h0, h1], axis=1)
        g1 = jnp.dot(a1, w1t, preferred_element_type=jnp.float32) + b1
        g0 = jnp.dot(h0, whh0t,
                     preferred_element_type=jnp.float32) + xg_ref[i * B:(i + 1) * B, :]

        i0, f0, gg0, o0 = _gates(g0, H)
        c0 = f0 * c0 + i0 * gg0
        h0 = o0 * jnp.tanh(c0)

        i1, f1, gg1, o1 = _gates(g1, H)
        c1 = f1 * c1 + i1 * gg1
        h1 = o1 * jnp.tanh(c1)
        out_ref[(i - 1) * B:i * B, :] = h1

    a1 = jnp.concatenate([h0, h1], axis=1)
    g1 = jnp.dot(a1, w1t, preferred_element_type=jnp.float32) + b1
    i1, f1, gg1, o1 = _gates(g1, H)
    c1 = f1 * c1 + i1 * gg1
    h1 = o1 * jnp.tanh(c1)
    out_ref[(S - 1) * B:S * B, :] = h1

    hn_ref[0, :, :] = h0
    hn_ref[1, :, :] = h1
    cn_ref[0, :, :] = c0
    cn_ref[1, :, :] = c1


def kernel(ids, embedding, w_ih_0, w_hh_0, b_ih_0, b_hh_0,
           w_ih_1, w_hh_1, b_ih_1, b_hh_1):
    B, S = ids.shape
    V, E = embedding.shape
    H = w_hh_0.shape[1]

    ids_col = jnp.transpose(ids).reshape(S * B, 1).astype(jnp.int32)

    out_shapes = (
        jax.ShapeDtypeStruct((S * B, H), jnp.float32),
        jax.ShapeDtypeStruct((2, B, H), jnp.float32),
        jax.ShapeDtypeStruct((2, B, H), jnp.float32),
    )

    def full(x):
        n = len(x.shape)
        return pl.BlockSpec(x.shape, lambda: (0,) * n)

    inputs = (ids_col, embedding, w_ih_0, w_hh_0,
              b_ih_0.reshape(1, 4 * H), b_hh_0.reshape(1, 4 * H),
              w_ih_1, w_hh_1,
              b_ih_1.reshape(1, 4 * H), b_hh_1.reshape(1, 4 * H))
    out_flat, h_n, c_n = pl.pallas_call(
        functools.partial(_encoder_kernel, seq_len=S, batch=B, hidden=H,
                          vocab=V),
        out_shape=out_shapes,
        in_specs=[full(x) for x in inputs],
        out_specs=tuple(pl.BlockSpec(s.shape, lambda n=len(s.shape): (0,) * n)
                        for s in out_shapes),
        scratch_shapes=[
            pltpu.VMEM((S * B, 4 * H), jnp.float32),
            pltpu.VMEM((H, 4 * H), jnp.float32),
            pltpu.VMEM((2 * H, 4 * H), jnp.float32),
        ],
        compiler_params=pltpu.CompilerParams(
            dimension_semantics=()),
    )(*inputs)

    out = jnp.transpose(out_flat.reshape(S, B, H), (1, 0, 2))
    return out, (h_n, c_n)


# speedup vs baseline: 1.6650x; 1.0920x over previous
import functools

import jax
import jax.numpy as jnp
from jax import lax
from jax.experimental import pallas as pl
from jax.experimental.pallas import tpu as pltpu


def _sig(x):
    return 1.0 / (1.0 + jnp.exp(-x))


def _gates(g, H):
    i_g = _sig(g[:, 0:H])
    f_g = _sig(g[:, H:2 * H])
    g_g = jnp.tanh(g[:, 2 * H:3 * H])
    o_g = _sig(g[:, 3 * H:4 * H])
    return i_g, f_g, g_g, o_g


def _encoder_kernel(ids_ref, emb_ref, wih0_ref, whh0_ref, bih0_ref, bhh0_ref,
                    wih1_ref, whh1_ref, bih1_ref, bhh1_ref,
                    out_ref, hn_ref, cn_ref,
                    xg_ref, whh0t_ref, w1t_ref,
                    *, seq_len, batch, hidden, vocab):
    S, B, H, V = seq_len, batch, hidden, vocab

    whh0t_ref[...] = jnp.transpose(whh0_ref[...])
    w1t_ref[0:H, :] = jnp.transpose(wih1_ref[...])
    w1t_ref[H:2 * H, :] = jnp.transpose(whh1_ref[...])
    b0 = bih0_ref[...] + bhh0_ref[...]
    b1 = bih1_ref[...] + bhh1_ref[...]

    emb = emb_ref[...]
    wih0t = jnp.transpose(wih0_ref[...])
    rows = S * B
    chunk = 128 if rows % 128 == 0 else rows
    tpc = chunk // B
    iota_bv = lax.broadcasted_iota(jnp.int32, (B, V), 1)
    for mc in range(rows // chunk):
        oh = jnp.concatenate(
            [(ids_ref[:, mc * tpc + tl:mc * tpc + tl + 1] == iota_bv)
             for tl in range(tpc)], axis=0).astype(jnp.float32)
        er = jnp.dot(oh, emb, preferred_element_type=jnp.float32)
        xg = jnp.dot(er, wih0t,
                     preferred_element_type=jnp.float32) + b0
        xg_ref[mc * chunk:(mc + 1) * chunk, :] = xg

    whh0t = whh0t_ref[...]
    w1t = w1t_ref[...]

    g0 = xg_ref[0:B, :]
    i0, f0, gg0, o0 = _gates(g0, H)
    c0 = i0 * gg0
    h0 = o0 * jnp.tanh(c0)
    h1 = jnp.zeros((B, H), jnp.float32)
    c1 = jnp.zeros((B, H), jnp.float32)

    for i in range(1, S):
        a1 = jnp.concatenate([h0, h1], axis=1)
        g1 = jnp.dot(a1, w1t, preferred_element_type=jnp.float32) + b1
        g0 = jnp.dot(h0, whh0t,
                     preferred_element_type=jnp.float32) + xg_ref[i * B:(i + 1) * B, :]

        i0, f0, gg0, o0 = _gates(g0, H)
        c0 = f0 * c0 + i0 * gg0
        h0 = o0 * jnp.tanh(c0)

        i1, f1, gg1, o1 = _gates(g1, H)
        c1 = f1 * c1 + i1 * gg1
        h1 = o1 * jnp.tanh(c1)
        out_ref[:, i - 1, :] = h1

    a1 = jnp.concatenate([h0, h1], axis=1)
    g1 = jnp.dot(a1, w1t, preferred_element_type=jnp.float32) + b1
    i1, f1, gg1, o1 = _gates(g1, H)
    c1 = f1 * c1 + i1 * gg1
    h1 = o1 * jnp.tanh(c1)
    out_ref[:, S - 1, :] = h1

    hn_ref[0, :, :] = h0
    hn_ref[1, :, :] = h1
    cn_ref[0, :, :] = c0
    cn_ref[1, :, :] = c1


def kernel(ids, embedding, w_ih_0, w_hh_0, b_ih_0, b_hh_0,
           w_ih_1, w_hh_1, b_ih_1, b_hh_1):
    B, S = ids.shape
    V, E = embedding.shape
    H = w_hh_0.shape[1]

    out_shapes = (
        jax.ShapeDtypeStruct((B, S, H), jnp.float32),
        jax.ShapeDtypeStruct((2, B, H), jnp.float32),
        jax.ShapeDtypeStruct((2, B, H), jnp.float32),
    )

    def full(x):
        n = len(x.shape)
        return pl.BlockSpec(x.shape, lambda: (0,) * n)

    inputs = (ids.astype(jnp.int32), embedding, w_ih_0, w_hh_0,
              b_ih_0.reshape(1, 4 * H), b_hh_0.reshape(1, 4 * H),
              w_ih_1, w_hh_1,
              b_ih_1.reshape(1, 4 * H), b_hh_1.reshape(1, 4 * H))
    out, h_n, c_n = pl.pallas_call(
        functools.partial(_encoder_kernel, seq_len=S, batch=B, hidden=H,
                          vocab=V),
        out_shape=out_shapes,
        in_specs=[full(x) for x in inputs],
        out_specs=tuple(pl.BlockSpec(s.shape, lambda n=len(s.shape): (0,) * n)
                        for s in out_shapes),
        scratch_shapes=[
            pltpu.VMEM((S * B, 4 * H), jnp.float32),
            pltpu.VMEM((H, 4 * H), jnp.float32),
            pltpu.VMEM((2 * H, 4 * H), jnp.float32),
        ],
        compiler_params=pltpu.CompilerParams(
            dimension_semantics=()),
    )(*inputs)

    return out, (h_n, c_n)


# speedup vs baseline: 1.6976x; 1.0196x over previous
import functools

import jax
import jax.numpy as jnp
from jax import lax
from jax.experimental import pallas as pl
from jax.experimental.pallas import tpu as pltpu


def _sig(x):
    return 1.0 / (1.0 + jnp.exp(-x))


def _gates(g, H):
    i_g = _sig(g[:, 0:H])
    f_g = _sig(g[:, H:2 * H])
    g_g = jnp.tanh(g[:, 2 * H:3 * H])
    o_g = _sig(g[:, 3 * H:4 * H])
    return i_g, f_g, g_g, o_g


def _encoder_kernel(ids_ref, emb_ref, wih0_ref, whh0_ref, bih0_ref, bhh0_ref,
                    wih1_ref, whh1_ref, bih1_ref, bhh1_ref,
                    out_ref, hn_ref, cn_ref,
                    xg_ref, whh0t_ref, w1t_ref,
                    wih0s_ref, whh0s_ref, wih1s_ref, whh1s_ref, sems,
                    *, seq_len, batch, hidden, vocab):
    S, B, H, V = seq_len, batch, hidden, vocab

    cp0 = pltpu.make_async_copy(wih0_ref, wih0s_ref, sems.at[0])
    cp1 = pltpu.make_async_copy(whh0_ref, whh0s_ref, sems.at[1])
    cp2 = pltpu.make_async_copy(wih1_ref, wih1s_ref, sems.at[2])
    cp3 = pltpu.make_async_copy(whh1_ref, whh1s_ref, sems.at[3])
    cp0.start()
    cp1.start()
    cp2.start()
    cp3.start()
    b0 = bih0_ref[...] + bhh0_ref[...]
    b1 = bih1_ref[...] + bhh1_ref[...]

    emb = emb_ref[...]
    cp0.wait()
    wih0t = jnp.transpose(wih0s_ref[...])
    rows = S * B
    chunk = 128 if rows % 128 == 0 else rows
    tpc = chunk // B
    iota_bv = lax.broadcasted_iota(jnp.int32, (B, V), 1)
    for mc in range(rows // chunk):
        oh = jnp.concatenate(
            [(ids_ref[:, mc * tpc + tl:mc * tpc + tl + 1] == iota_bv)
             for tl in range(tpc)], axis=0).astype(jnp.float32)
        er = jnp.dot(oh, emb, preferred_element_type=jnp.float32)
        xg = jnp.dot(er, wih0t,
                     preferred_element_type=jnp.float32) + b0
        xg_ref[mc * chunk:(mc + 1) * chunk, :] = xg

    cp1.wait()
    whh0t_ref[...] = jnp.transpose(whh0s_ref[...])
    cp2.wait()
    w1t_ref[0:H, :] = jnp.transpose(wih1s_ref[...])
    cp3.wait()
    w1t_ref[H:2 * H, :] = jnp.transpose(whh1s_ref[...])

    whh0t = whh0t_ref[...]
    w1t = w1t_ref[...]

    g0 = xg_ref[0:B, :]
    i0, f0, gg0, o0 = _gates(g0, H)
    c0 = i0 * gg0
    h0 = o0 * jnp.tanh(c0)
    h1 = jnp.zeros((B, H), jnp.float32)
    c1 = jnp.zeros((B, H), jnp.float32)

    for i in range(1, S):
        a1 = jnp.concatenate([h0, h1], axis=1)
        g1 = jnp.dot(a1, w1t, preferred_element_type=jnp.float32) + b1
        g0 = jnp.dot(h0, whh0t,
                     preferred_element_type=jnp.float32) + xg_ref[i * B:(i + 1) * B, :]

        i0, f0, gg0, o0 = _gates(g0, H)
        c0 = f0 * c0 + i0 * gg0
        h0 = o0 * jnp.tanh(c0)

        i1, f1, gg1, o1 = _gates(g1, H)
        c1 = f1 * c1 + i1 * gg1
        h1 = o1 * jnp.tanh(c1)
        out_ref[:, i - 1, :] = h1

    a1 = jnp.concatenate([h0, h1], axis=1)
    g1 = jnp.dot(a1, w1t, preferred_element_type=jnp.float32) + b1
    i1, f1, gg1, o1 = _gates(g1, H)
    c1 = f1 * c1 + i1 * gg1
    h1 = o1 * jnp.tanh(c1)
    out_ref[:, S - 1, :] = h1

    hn_ref[0, :, :] = h0
    hn_ref[1, :, :] = h1
    cn_ref[0, :, :] = c0
    cn_ref[1, :, :] = c1


def kernel(ids, embedding, w_ih_0, w_hh_0, b_ih_0, b_hh_0,
           w_ih_1, w_hh_1, b_ih_1, b_hh_1):
    B, S = ids.shape
    V, E = embedding.shape
    H = w_hh_0.shape[1]

    out_shapes = (
        jax.ShapeDtypeStruct((B, S, H), jnp.float32),
        jax.ShapeDtypeStruct((2, B, H), jnp.float32),
        jax.ShapeDtypeStruct((2, B, H), jnp.float32),
    )

    def full(x):
        n = len(x.shape)
        return pl.BlockSpec(x.shape, lambda: (0,) * n)

    inputs = (ids.astype(jnp.int32), embedding, w_ih_0, w_hh_0,
              b_ih_0.reshape(1, 4 * H), b_hh_0.reshape(1, 4 * H),
              w_ih_1, w_hh_1,
              b_ih_1.reshape(1, 4 * H), b_hh_1.reshape(1, 4 * H))
    hbm = frozenset([2, 3, 6, 7])
    in_specs = [pl.BlockSpec(memory_space=pl.ANY) if i in hbm else full(x)
                for i, x in enumerate(inputs)]
    out, h_n, c_n = pl.pallas_call(
        functools.partial(_encoder_kernel, seq_len=S, batch=B, hidden=H,
                          vocab=V),
        out_shape=out_shapes,
        in_specs=in_specs,
        out_specs=tuple(pl.BlockSpec(s.shape, lambda n=len(s.shape): (0,) * n)
                        for s in out_shapes),
        scratch_shapes=[
            pltpu.VMEM((S * B, 4 * H), jnp.float32),
            pltpu.VMEM((H, 4 * H), jnp.float32),
            pltpu.VMEM((2 * H, 4 * H), jnp.float32),
            pltpu.VMEM((4 * H, E), jnp.float32),
            pltpu.VMEM((4 * H, H), jnp.float32),
            pltpu.VMEM((4 * H, H), jnp.float32),
            pltpu.VMEM((4 * H, H), jnp.float32),
            pltpu.SemaphoreType.DMA((4,)),
        ],
        compiler_params=pltpu.CompilerParams(
            dimension_semantics=()),
    )(*inputs)

    return out, (h_n, c_n)


# speedup vs baseline: 1.8345x; 1.0806x over previous
import functools

import jax
import jax.numpy as jnp
from jax import lax
from jax.experimental import pallas as pl
from jax.experimental.pallas import tpu as pltpu


def _sig(x):
    return 1.0 / (1.0 + jnp.exp(-x))


def _gates(g, H):
    i_g = _sig(g[:, 0:H])
    f_g = _sig(g[:, H:2 * H])
    g_g = jnp.tanh(g[:, 2 * H:3 * H])
    o_g = _sig(g[:, 3 * H:4 * H])
    return i_g, f_g, g_g, o_g


def _encoder_kernel(ids_ref, emb_ref, wih0_ref, whh0_ref, bih0_ref, bhh0_ref,
                    wih1_ref, whh1_ref, bih1_ref, bhh1_ref,
                    out_ref, hn_ref, cn_ref,
                    xg_ref, whh0t_ref, w1t_ref,
                    wih0s_ref, whh0s_ref, wih1s_ref, whh1s_ref, sems,
                    *, seq_len, batch, hidden, vocab):
    S, B, H, V = seq_len, batch, hidden, vocab

    cp0 = pltpu.make_async_copy(wih0_ref, wih0s_ref, sems.at[0])
    cp1 = pltpu.make_async_copy(whh0_ref, whh0s_ref, sems.at[1])
    cp2 = pltpu.make_async_copy(wih1_ref, wih1s_ref, sems.at[2])
    cp3 = pltpu.make_async_copy(whh1_ref, whh1s_ref, sems.at[3])
    cp0.start()
    cp1.start()
    cp2.start()
    cp3.start()
    b0 = bih0_ref[...] + bhh0_ref[...]
    b1 = bih1_ref[...] + bhh1_ref[...]

    emb = emb_ref[...]
    cp0.wait()
    wih0t = jnp.transpose(wih0s_ref[...])
    rows = S * B
    chunk = 128 if rows % 128 == 0 else rows
    tpc = chunk // B
    iota_bv = lax.broadcasted_iota(jnp.int32, (B, V), 1)
    for mc in range(rows // chunk):
        oh = jnp.concatenate(
            [(ids_ref[:, mc * tpc + tl:mc * tpc + tl + 1] == iota_bv)
             for tl in range(tpc)], axis=0).astype(jnp.float32)
        er = jnp.dot(oh, emb, preferred_element_type=jnp.float32)
        xg = jnp.dot(er, wih0t,
                     preferred_element_type=jnp.float32) + b0
        xg_ref[mc * chunk:(mc + 1) * chunk, :] = xg

    cp1.wait()
    whh0t_ref[...] = jnp.transpose(whh0s_ref[...])
    cp2.wait()
    w1t_ref[0:H, :] = jnp.transpose(wih1s_ref[...])
    cp3.wait()
    w1t_ref[H:2 * H, :] = jnp.transpose(whh1s_ref[...])

    whh0t = whh0t_ref[...]
    w1t = w1t_ref[...]

    g0 = xg_ref[0:B, :]
    i0, f0, gg0, o0 = _gates(g0, H)
    c0 = i0 * gg0
    h0m2 = o0 * jnp.tanh(c0)
    g0 = jnp.dot(h0m2, whh0t,
                 preferred_element_type=jnp.float32) + xg_ref[B:2 * B, :]
    i0, f0, gg0, o0 = _gates(g0, H)
    c0 = f0 * c0 + i0 * gg0
    h0m1 = o0 * jnp.tanh(c0)
    h1 = jnp.zeros((B, H), jnp.float32)
    c1 = jnp.zeros((B, H), jnp.float32)

    def l1_step(h0_in, h1_in, c1_in, t_out):
        a1 = jnp.concatenate([h0_in, h1_in], axis=1)
        g1 = jnp.dot(a1, w1t, preferred_element_type=jnp.float32) + b1
        i1, f1, gg1, o1 = _gates(g1, H)
        c1n = f1 * c1_in + i1 * gg1
        h1n = o1 * jnp.tanh(c1n)
        out_ref[:, t_out, :] = h1n
        return h1n, c1n

    for i in range(2, S):
        h1, c1 = l1_step(h0m2, h1, c1, i - 2)
        g0 = jnp.dot(h0m1, whh0t,
                     preferred_element_type=jnp.float32) + xg_ref[i * B:(i + 1) * B, :]
        i0, f0, gg0, o0 = _gates(g0, H)
        c0 = f0 * c0 + i0 * gg0
        h0m2 = h0m1
        h0m1 = o0 * jnp.tanh(c0)

    h1, c1 = l1_step(h0m2, h1, c1, S - 2)
    h1, c1 = l1_step(h0m1, h1, c1, S - 1)
    h0 = h0m1

    hn_ref[0, :, :] = h0
    hn_ref[1, :, :] = h1
    cn_ref[0, :, :] = c0
    cn_ref[1, :, :] = c1


def kernel(ids, embedding, w_ih_0, w_hh_0, b_ih_0, b_hh_0,
           w_ih_1, w_hh_1, b_ih_1, b_hh_1):
    B, S = ids.shape
    V, E = embedding.shape
    H = w_hh_0.shape[1]

    out_shapes = (
        jax.ShapeDtypeStruct((B, S, H), jnp.float32),
        jax.ShapeDtypeStruct((2, B, H), jnp.float32),
        jax.ShapeDtypeStruct((2, B, H), jnp.float32),
    )

    def full(x):
        n = len(x.shape)
        return pl.BlockSpec(x.shape, lambda: (0,) * n)

    inputs = (ids.astype(jnp.int32), embedding, w_ih_0, w_hh_0,
              b_ih_0.reshape(1, 4 * H), b_hh_0.reshape(1, 4 * H),
              w_ih_1, w_hh_1,
              b_ih_1.reshape(1, 4 * H), b_hh_1.reshape(1, 4 * H))
    hbm = frozenset([2, 3, 6, 7])
    in_specs = [pl.BlockSpec(memory_space=pl.ANY) if i in hbm else full(x)
                for i, x in enumerate(inputs)]
    out, h_n, c_n = pl.pallas_call(
        functools.partial(_encoder_kernel, seq_len=S, batch=B, hidden=H,
                          vocab=V),
        out_shape=out_shapes,
        in_specs=in_specs,
        out_specs=tuple(pl.BlockSpec(s.shape, lambda n=len(s.shape): (0,) * n)
                        for s in out_shapes),
        scratch_shapes=[
            pltpu.VMEM((S * B, 4 * H), jnp.float32),
            pltpu.VMEM((H, 4 * H), jnp.float32),
            pltpu.VMEM((2 * H, 4 * H), jnp.float32),
            pltpu.VMEM((4 * H, E), jnp.float32),
            pltpu.VMEM((4 * H, H), jnp.float32),
            pltpu.VMEM((4 * H, H), jnp.float32),
            pltpu.VMEM((4 * H, H), jnp.float32),
            pltpu.SemaphoreType.DMA((4,)),
        ],
        compiler_params=pltpu.CompilerParams(
            dimension_semantics=()),
    )(*inputs)

    return out, (h_n, c_n)
